# baseline (speedup 1.0000x reference)
import functools
import math

import jax
import jax.numpy as jnp
from jax.experimental import pallas as pl
from jax.experimental.pallas import tpu as pltpu

_INV_SQRT2 = 1.0 / math.sqrt(2.0)


def _router_kernel(block_n, nbuf, x_hbm, w1_ref, b1_ref, w2_ref, b2_ref,
                   packets_ref, probs_ref, xbuf, dma_sems):
    n = x_hbm.shape[0]
    nblocks = n // block_n
    w1 = w1_ref[...]
    b1 = b1_ref[...]
    w2 = w2_ref[...]
    b2 = b2_ref[...]

    def start_copy(blk, slot):
        pltpu.make_async_copy(
            x_hbm.at[pl.ds(blk * block_n, block_n), :],
            xbuf.at[slot],
            dma_sems.at[slot],
        ).start()

    for s in range(nbuf - 1):
        start_copy(s, s)

    def step(i, carry):
        refill_blk = i + nbuf - 1
        @pl.when(refill_blk < nblocks)
        def _():
            start_copy(refill_blk, refill_blk % nbuf)
        slot = jax.lax.rem(i, nbuf)
        pltpu.make_async_copy(
            x_hbm.at[pl.ds(i * block_n, block_n), :],
            xbuf.at[slot],
            dma_sems.at[slot],
        ).wait()
        h = jnp.dot(xbuf[slot], w1, preferred_element_type=jnp.float32)
        h = h + b1
        h = 0.5 * h * (1.0 + jax.lax.erf(h * _INV_SQRT2))
        logits = jnp.dot(h, w2, preferred_element_type=jnp.float32)
        logits = logits + b2
        row = pl.ds(i * block_n, block_n)
        packets_ref[row, :] = jnp.argmax(
            logits, axis=-1, keepdims=True).astype(jnp.int32)
        m = jnp.max(logits, axis=-1, keepdims=True)
        e = jnp.exp(logits - m)
        probs_ref[row, :] = e / jnp.sum(e, axis=-1, keepdims=True)
        return carry

    jax.lax.fori_loop(0, nblocks, step, 0)


@functools.partial(jax.jit, static_argnames=("block_n", "nbuf"))
def kernel(x, W1, b1, W2, b2, block_n: int = 512, nbuf: int = 10):
    n, d = x.shape
    h_dim = W1.shape[1]
    p = W2.shape[1]
    packets2d, probs = pl.pallas_call(
        functools.partial(_router_kernel, block_n, nbuf),
        in_specs=[
            pl.BlockSpec(memory_space=pltpu.MemorySpace.HBM),
            pl.BlockSpec(memory_space=pltpu.MemorySpace.VMEM),
            pl.BlockSpec(memory_space=pltpu.MemorySpace.VMEM),
            pl.BlockSpec(memory_space=pltpu.MemorySpace.VMEM),
            pl.BlockSpec(memory_space=pltpu.MemorySpace.VMEM),
        ],
        out_specs=[
            pl.BlockSpec(memory_space=pltpu.MemorySpace.VMEM),
            pl.BlockSpec(memory_space=pltpu.MemorySpace.VMEM),
        ],
        out_shape=[
            jax.ShapeDtypeStruct((n, 1), jnp.int32),
            jax.ShapeDtypeStruct((n, p), jnp.float32),
        ],
        scratch_shapes=[
            pltpu.VMEM((nbuf, block_n, d), jnp.float32),
            pltpu.SemaphoreType.DMA((nbuf,)),
        ],
    )(x, W1, b1, W2, b2)
    return packets2d.reshape(n), probs


# speedup vs baseline: 1.2689x; 1.2689x over previous
import functools
import math

import jax
import jax.numpy as jnp
from jax.experimental import pallas as pl
from jax.experimental.pallas import tpu as pltpu

_INV_SQRT2 = 1.0 / math.sqrt(2.0)


def _router_kernel(block_n, nbuf, x_hbm, w1_ref, b1_ref, w2_ref, b2_ref,
                   packets_ref, probs_ref, xbuf, dma_sems):
    i = pl.program_id(0)
    nblocks = pl.num_programs(0)

    def start_copy(blk, slot):
        pltpu.make_async_copy(
            x_hbm.at[pl.ds(blk * block_n, block_n), :],
            xbuf.at[slot],
            dma_sems.at[slot],
        ).start()

    @pl.when(i == 0)
    def _():
        for s in range(nbuf - 1):
            start_copy(s, s)

    refill_blk = i + nbuf - 1

    @pl.when(refill_blk < nblocks)
    def _():
        start_copy(refill_blk, refill_blk % nbuf)

    slot = jax.lax.rem(i, nbuf)
    pltpu.make_async_copy(
        x_hbm.at[pl.ds(i * block_n, block_n), :],
        xbuf.at[slot],
        dma_sems.at[slot],
    ).wait()

    h = jnp.dot(xbuf[slot], w1_ref[...], preferred_element_type=jnp.float32)
    h = h + b1_ref[...]
    h = 0.5 * h * (1.0 + jax.lax.erf(h * _INV_SQRT2))
    logits = jnp.dot(h, w2_ref[...], preferred_element_type=jnp.float32)
    logits = logits + b2_ref[...]
    packets_ref[...] = jnp.argmax(
        logits, axis=-1, keepdims=True).astype(jnp.int32)
    m = jnp.max(logits, axis=-1, keepdims=True)
    e = jnp.exp(logits - m)
    probs_ref[...] = e / jnp.sum(e, axis=-1, keepdims=True)


@functools.partial(jax.jit, static_argnames=("block_n", "nbuf"))
def kernel(x, W1, b1, W2, b2, block_n: int = 2048, nbuf: int = 6):
    n, d = x.shape
    h_dim = W1.shape[1]
    p = W2.shape[1]
    nblocks = n // block_n
    packets2d, probs = pl.pallas_call(
        functools.partial(_router_kernel, block_n, nbuf),
        grid=(nblocks,),
        in_specs=[
            pl.BlockSpec(memory_space=pltpu.MemorySpace.HBM),
            pl.BlockSpec((d, h_dim), lambda i: (0, 0)),
            pl.BlockSpec((h_dim,), lambda i: (0,)),
            pl.BlockSpec((h_dim, p), lambda i: (0, 0)),
            pl.BlockSpec((p,), lambda i: (0,)),
        ],
        out_specs=[
            pl.BlockSpec((block_n, 1), lambda i: (i, 0)),
            pl.BlockSpec((block_n, p), lambda i: (i, 0)),
        ],
        out_shape=[
            jax.ShapeDtypeStruct((n, 1), jnp.int32),
            jax.ShapeDtypeStruct((n, p), jnp.float32),
        ],
        scratch_shapes=[
            pltpu.VMEM((nbuf, block_n, d), jnp.float32),
            pltpu.SemaphoreType.DMA((nbuf,)),
        ],
        compiler_params=pltpu.CompilerParams(
            dimension_semantics=("arbitrary",),
        ),
    )(x, W1, b1, W2, b2)
    return packets2d.reshape(n), probs
